# Initial kernel scaffold; baseline (speedup 1.0000x reference)
import jax
import jax.numpy as jnp
from jax.experimental import pallas as pl


def kernel(x, fc_weight, bias):
    raise NotImplementedError("write your pallas kernel here")


# speedup vs baseline: 1.2049x; 1.2049x over previous
import functools

import jax
import jax.numpy as jnp
from jax import lax
from jax.experimental import pallas as pl
from jax.experimental.pallas import tpu as pltpu
from jax.experimental.pallas import tpu_sc as plsc

_LANES = 16


def _make_sc_kernel(batch, num_fields, nc, ns):
    nw = nc * ns
    n_per = batch // nw
    n_flat = n_per * num_fields

    def body(x_hbm, w_hbm, b_hbm, out_hbm, idx_v, vals_v, out_v, bias_v, sem):
        cid = lax.axis_index("c")
        sid = lax.axis_index("s")
        wid = sid * nc + cid
        base = wid * n_flat

        pltpu.sync_copy(x_hbm.at[pl.ds(base, n_flat)], idx_v)
        pltpu.sync_copy(b_hbm, bias_v.at[pl.ds(0, 1)])

        nchunks = n_flat // 128

        def gstep(s, carry):
            descs = []
            for k in range(8):
                sl = pl.ds((s * 8 + k) * 128, 128)
                descs.append(
                    pltpu.async_copy(w_hbm.at[idx_v.at[sl]], vals_v.at[sl], sem)
                )
            for d in descs:
                d.wait()
            return carry

        lax.fori_loop(0, nchunks // 8, gstep, 0)

        bias_vec = jnp.broadcast_to(bias_v[pl.ds(0, _LANES)][0], (_LANES,))
        lane_f = lax.iota(jnp.int32, _LANES) * num_fields

        def chunk(c, carry):
            g0 = lane_f + c * (_LANES * num_fields)
            acc = bias_vec
            for j in range(num_fields):
                acc = acc + plsc.load_gather(vals_v, [g0 + j])
            out_v[pl.ds(c * _LANES, _LANES)] = acc
            return carry

        lax.fori_loop(0, n_per // _LANES, chunk, 0)
        pltpu.sync_copy(out_v, out_hbm.at[pl.ds(wid * n_per, n_per)])

    mesh = plsc.VectorSubcoreMesh(core_axis_name="c", subcore_axis_name="s")
    return pl.kernel(
        body,
        out_type=jax.ShapeDtypeStruct((batch,), jnp.float32),
        mesh=mesh,
        compiler_params=pltpu.CompilerParams(needs_layout_passes=False),
        scratch_types=[
            pltpu.VMEM((n_flat,), jnp.int32),
            pltpu.VMEM((n_flat,), jnp.float32),
            pltpu.VMEM((n_per,), jnp.float32),
            pltpu.VMEM((128,), jnp.float32),
            pltpu.SemaphoreType.DMA,
        ],
    )


@jax.jit
def kernel(x, fc_weight, bias):
    batch, num_fields = x.shape
    info = plsc.get_sparse_core_info()
    nc, ns = info.num_cores, info.num_subcores

    x_flat = x.astype(jnp.int32).reshape(-1)
    w_flat = fc_weight.reshape(-1)
    b_flat = bias.reshape(-1).astype(jnp.float32)

    sc = _make_sc_kernel(batch, num_fields, nc, ns)
    out = sc(x_flat, w_flat, b_flat)
    return out.reshape(batch, 1)


# speedup vs baseline: 1.3208x; 1.0962x over previous
import functools

import jax
import jax.numpy as jnp
from jax import lax
from jax.experimental import pallas as pl
from jax.experimental.pallas import tpu as pltpu
from jax.experimental.pallas import tpu_sc as plsc

_LANES = 16


def _make_sc_kernel(batch, num_fields, nc, ns):
    nw = nc * ns
    n_per = batch // nw
    n_flat = n_per * num_fields

    def body(x_hbm, w_hbm, b_hbm, out_hbm, idx_v, vals_v, out_v, bias_v, sem):
        cid = lax.axis_index("c")
        sid = lax.axis_index("s")
        wid = sid * nc + cid
        base = wid * n_flat

        pltpu.sync_copy(x_hbm.at[pl.ds(base, n_flat)], idx_v)
        pltpu.sync_copy(b_hbm, bias_v.at[pl.ds(0, 1)])

        pltpu.async_copy(w_hbm.at[idx_v], vals_v, sem).wait()

        bias_vec = jnp.broadcast_to(bias_v[pl.ds(0, _LANES)][0], (_LANES,))
        lane_f = lax.iota(jnp.int32, _LANES) * num_fields

        def chunk(c, carry):
            g0 = lane_f + c * (_LANES * num_fields)
            acc = bias_vec
            for j in range(num_fields):
                acc = acc + plsc.load_gather(vals_v, [g0 + j])
            out_v[pl.ds(c * _LANES, _LANES)] = acc
            return carry

        lax.fori_loop(0, n_per // _LANES, chunk, 0)
        pltpu.sync_copy(out_v, out_hbm.at[pl.ds(wid * n_per, n_per)])

    mesh = plsc.VectorSubcoreMesh(core_axis_name="c", subcore_axis_name="s")
    return pl.kernel(
        body,
        out_type=jax.ShapeDtypeStruct((batch,), jnp.float32),
        mesh=mesh,
        compiler_params=pltpu.CompilerParams(needs_layout_passes=False),
        scratch_types=[
            pltpu.VMEM((n_flat,), jnp.int32),
            pltpu.VMEM((n_flat,), jnp.float32),
            pltpu.VMEM((n_per,), jnp.float32),
            pltpu.VMEM((128,), jnp.float32),
            pltpu.SemaphoreType.DMA,
        ],
    )


@jax.jit
def kernel(x, fc_weight, bias):
    batch, num_fields = x.shape
    info = plsc.get_sparse_core_info()
    nc, ns = info.num_cores, info.num_subcores

    x_flat = x.astype(jnp.int32).reshape(-1)
    w_flat = fc_weight.reshape(-1)
    b_flat = bias.reshape(-1).astype(jnp.float32)

    sc = _make_sc_kernel(batch, num_fields, nc, ns)
    out = sc(x_flat, w_flat, b_flat)
    return out.reshape(batch, 1)
